# Initial kernel scaffold; baseline (speedup 1.0000x reference)
import jax
import jax.numpy as jnp
from jax.experimental import pallas as pl


def kernel(x, edge_index, edge_attr, batch, W_l1, W_r1, b_l1, W_l2, W_r2, b_l2, W_blk, b_blk, W1, b1, W2, b2):
    raise NotImplementedError("write your pallas kernel here")


# speedup vs baseline: 3.0998x; 3.0998x over previous
import functools

import jax
import jax.numpy as jnp
from jax import lax
from jax.experimental import pallas as pl
from jax.experimental.pallas import tpu as pltpu
from jax.experimental.pallas import tpu_sc as plsc

_NC = 2
_NS = 16
_NW = _NC * _NS
_K = 128
_NUM_GRAPHS = 64


def _ceil_to(a, m):
    return (a + m - 1) // m * m


_G = 8


def _make_sc_scatter(n_pad, e_pad, d, with_deg):
    t_per_w = e_pad // _NW
    steps = t_per_w // _K
    groups = steps // _G
    rows_per_sub = n_pad // _NS
    hr = n_pad // _K
    hrp = _ceil_to(hr, 8)

    mesh = plsc.VectorSubcoreMesh(core_axis_name="c", subcore_axis_name="s")

    out_type = [jax.ShapeDtypeStruct((_NC * n_pad, d), jnp.float32)]
    scratch = [
        pltpu.VMEM((_G, _K), jnp.int32),
        pltpu.VMEM((_G, _K), jnp.int32),
        pltpu.VMEM((_G, _K), jnp.int32),
        pltpu.VMEM((_G, _K), jnp.int32),
        pltpu.VMEM((_K, d), jnp.float32),
        pltpu.VMEM((_K, d), jnp.float32),
        pltpu.SemaphoreType.DMA,
        pltpu.SemaphoreType.DMA,
        pltpu.VMEM_SHARED((n_pad, d), jnp.float32),
    ]
    if with_deg:
        out_type.append(jax.ShapeDtypeStruct((_NC * hrp, _K), jnp.float32))
        scratch += [
            pltpu.VMEM((hrp, _K), jnp.float32),
            pltpu.VMEM((hr,), jnp.int32),
            pltpu.VMEM_SHARED((hrp, _K), jnp.float32),
        ]

    @functools.partial(
        pl.kernel, mesh=mesh, out_type=out_type, scratch_types=scratch,
        compiler_params=pltpu.CompilerParams(needs_layout_passes=False))
    def sc_kernel(x_hbm, src_hbm, dst_hbm, z_hbm, rix_hbm, *refs):
        if with_deg:
            (acc_out, deg_out, src0_v, src1_v, dst0_v, dst1_v, rows0_v,
             rows1_v, sem0, sem1, acc_sh, hist_v, rix_v, deg_sh) = refs
        else:
            (acc_out, src0_v, src1_v, dst0_v, dst1_v, rows0_v, rows1_v,
             sem0, sem1, acc_sh) = refs
        srcs, dsts = (src0_v, src1_v), (dst0_v, dst1_v)
        rows, sems = (rows0_v, rows1_v), (sem0, sem1)
        cid = lax.axis_index("c")
        sid = lax.axis_index("s")
        wid = sid * _NC + cid

        r0 = sid * rows_per_sub
        pltpu.sync_copy(z_hbm.at[pl.ds(r0, rows_per_sub)],
                        acc_sh.at[pl.ds(r0, rows_per_sub)])
        if with_deg:
            pltpu.sync_copy(z_hbm.at[pl.ds(0, hrp)], hist_v)
            pltpu.sync_copy(rix_hbm, rix_v)

            @pl.when(sid == 0)
            def _():
                pltpu.sync_copy(z_hbm.at[pl.ds(0, hrp)], deg_sh)

        plsc.subcore_barrier()

        base_row = wid * steps
        ones16 = jnp.ones((16,), jnp.float32)

        def load_group(g, b):
            gr = base_row + lax.rem(g * _G, steps)
            pltpu.sync_copy(src_hbm.at[pl.ds(gr, _G)], srcs[b])
            pltpu.sync_copy(dst_hbm.at[pl.ds(gr, _G)], dsts[b])

        load_group(0, 0)
        pltpu.async_copy(x_hbm.at[srcs[0].at[0]], rows[0], sems[0])

        def pair(gp, carry):
            for gb in range(2):
                g = gp * 2 + gb
                load_group(g + 1, 1 - gb)
                for j in range(_G):
                    cur = j % 2
                    nxt = 1 - cur
                    if j < _G - 1:
                        nidx = srcs[gb].at[j + 1]
                    else:
                        nidx = srcs[1 - gb].at[0]
                    pltpu.async_copy(x_hbm.at[nidx], rows[nxt], sems[nxt])
                    pltpu.make_async_copy(
                        x_hbm.at[srcs[gb].at[j]], rows[cur], sems[cur]).wait()
                    pltpu.sync_copy(rows[cur], acc_sh.at[dsts[gb].at[j]],
                                    add=True)
                    if with_deg:
                        for q in range(_K // 16):
                            dvec = dsts[gb][j, pl.ds(q * 16, 16)]
                            ridx = lax.shift_right_logical(dvec, 7)
                            cidx = lax.bitwise_and(dvec, 127)
                            plsc.addupdate_scatter(hist_v, [ridx, cidx], ones16)
            return carry

        lax.fori_loop(0, groups // 2, pair, 0)
        pltpu.make_async_copy(x_hbm.at[srcs[0].at[0]], rows[0], sems[0]).wait()
        if with_deg:
            pltpu.sync_copy(hist_v.at[pl.ds(0, hr)], deg_sh.at[rix_v],
                            add=True)
        plsc.subcore_barrier()

        o0 = cid * n_pad + r0
        pltpu.sync_copy(acc_sh.at[pl.ds(r0, rows_per_sub)],
                        acc_out.at[pl.ds(o0, rows_per_sub)])
        if with_deg:
            @pl.when(sid == 0)
            def _():
                pltpu.sync_copy(deg_sh, deg_out.at[pl.ds(cid * hrp, hrp)])

    return sc_kernel


def _sage_linear(p, deg, x, wl, wr, b, n_pad, br):
    nsteps = n_pad // br
    h = wl.shape[1]
    d = x.shape[1]

    def body(p0_r, p1_r, deg_r, x_r, wl_r, wr_r, b_r, o_r):
        s = (p0_r[...] + p1_r[...]) / deg_r[...]
        acc = (jnp.dot(s, wl_r[...], preferred_element_type=jnp.float32)
               + jnp.dot(x_r[...], wr_r[...], preferred_element_type=jnp.float32)
               + b_r[...])
        o_r[...] = jnp.maximum(acc, 0.0)

    full = lambda a: pl.BlockSpec(a.shape, lambda i: (0,) * a.ndim)
    return pl.pallas_call(
        body,
        grid=(nsteps,),
        in_specs=[
            pl.BlockSpec((br, d), lambda i: (i, 0)),
            pl.BlockSpec((br, d), lambda i: (i + nsteps, 0)),
            pl.BlockSpec((br, 1), lambda i: (i, 0)),
            pl.BlockSpec((br, d), lambda i: (i, 0)),
            full(wl), full(wr), full(b),
        ],
        out_specs=pl.BlockSpec((br, h), lambda i: (i, 0)),
        out_shape=jax.ShapeDtypeStruct((n_pad, h), jnp.float32),
    )(p, p, deg, x, wl, wr, b)


def _pool_stage(p, deg, x1, batch3d, wl2, wr2, b2, wbt, wbb, bb, n_pad, br):
    nsteps = n_pad // br
    g = _NUM_GRAPHS

    def body(p0_r, p1_r, deg_r, x1_r, bat_r, wl_r, wr_r, b_r,
             wbt_r, wbb_r, bb_r, ps_r, cnt_r):
        i = pl.program_id(0)
        s = (p0_r[...] + p1_r[...]) / deg_r[...]
        x2 = jnp.maximum(
            jnp.dot(s, wl_r[...], preferred_element_type=jnp.float32)
            + jnp.dot(x1_r[...], wr_r[...], preferred_element_type=jnp.float32)
            + b_r[...], 0.0)
        hb = jnp.maximum(
            jnp.dot(x1_r[...], wbt_r[...], preferred_element_type=jnp.float32)
            + jnp.dot(x2, wbb_r[...], preferred_element_type=jnp.float32)
            + bb_r[...], 0.0)
        bids = bat_r[0, 0, :]
        onehot = (lax.broadcasted_iota(jnp.int32, (g, br), 0)
                  == bids[None, :]).astype(jnp.float32)
        ps = jnp.dot(onehot, hb, preferred_element_type=jnp.float32)
        cs = jnp.broadcast_to(jnp.sum(onehot, axis=1, keepdims=True),
                              (g, hb.shape[1]))

        @pl.when(i == 0)
        def _():
            ps_r[...] = ps
            cnt_r[...] = cs

        @pl.when(i > 0)
        def _():
            ps_r[...] += ps
            cnt_r[...] += cs

    d = x1.shape[1]
    full = lambda a: pl.BlockSpec(a.shape, lambda i: (0,) * a.ndim)
    return pl.pallas_call(
        body,
        grid=(nsteps,),
        in_specs=[
            pl.BlockSpec((br, d), lambda i: (i, 0)),
            pl.BlockSpec((br, d), lambda i: (i + nsteps, 0)),
            pl.BlockSpec((br, 1), lambda i: (i, 0)),
            pl.BlockSpec((br, d), lambda i: (i, 0)),
            pl.BlockSpec((1, 1, br), lambda i: (i, 0, 0)),
            full(wl2), full(wr2), full(b2), full(wbt), full(wbb), full(bb),
        ],
        out_specs=[
            pl.BlockSpec((g, d), lambda i: (0, 0)),
            pl.BlockSpec((g, d), lambda i: (0, 0)),
        ],
        out_shape=[
            jax.ShapeDtypeStruct((g, d), jnp.float32),
            jax.ShapeDtypeStruct((g, d), jnp.float32),
        ],
    )(p, p, deg, x1, batch3d, wl2, wr2, b2, wbt, wbb, bb)


def _head(ps, cnt, w1, b1, w2, b2):
    g, d = ps.shape
    c = w2.shape[1]

    def body(ps_r, cnt_r, w1_r, b1_r, w2_r, b2_r, o_r):
        pooled = ps_r[...] / jnp.maximum(cnt_r[...], 1.0)
        t = jnp.maximum(
            jnp.dot(pooled, w1_r[...], preferred_element_type=jnp.float32)
            + b1_r[...], 0.0)
        z = (jnp.dot(t, w2_r[...], preferred_element_type=jnp.float32)
             + b2_r[...])
        m = jnp.max(z, axis=-1, keepdims=True)
        e = jnp.exp(z - m)
        lse = jnp.log(jnp.sum(e, axis=-1, keepdims=True)) + m
        o_r[...] = z - lse

    full = lambda a: pl.BlockSpec(a.shape, lambda: (0,) * a.ndim)
    return pl.pallas_call(
        body,
        in_specs=[full(ps), full(cnt), full(w1), full(b1), full(w2), full(b2)],
        out_specs=pl.BlockSpec((g, c), lambda: (0, 0)),
        out_shape=jax.ShapeDtypeStruct((g, c), jnp.float32),
    )(ps, cnt, w1, b1, w2, b2)


def kernel(x, edge_index, edge_attr, batch, W_l1, W_r1, b_l1, W_l2, W_r2,
           b_l2, W_blk, b_blk, W1, b1, W2, b2):
    n, d = x.shape
    e = edge_index.shape[1]
    h = W_l1.shape[1]

    br = 1264
    n_pad = _ceil_to(n + 1, _K)
    n_pad = _ceil_to(n_pad, br)
    e_pad = _ceil_to(e, _NW * _K * _G * 2)
    hr = n_pad // _K
    hrp = _ceil_to(hr, 8)

    src = jnp.pad(edge_index[0], (0, e_pad - e)).reshape(e_pad // _K, _K)
    dst = jnp.pad(edge_index[1], (0, e_pad - e),
                  constant_values=n).reshape(e_pad // _K, _K)
    x_p = jnp.pad(x, ((0, n_pad - n), (0, 0)))
    batch_p = jnp.pad(batch, (0, n_pad - n), constant_values=_NUM_GRAPHS)
    batch3d = batch_p.reshape(n_pad // br, 1, br)

    z128 = jnp.zeros((n_pad, d), jnp.float32)
    rix = jnp.arange(hr, dtype=jnp.int32)

    p1, dpart = _make_sc_scatter(n_pad, e_pad, d, True)(
        x_p, src, dst, z128, rix)
    deg = (dpart[:hrp] + dpart[hrp:]).reshape(-1)[:n_pad]
    deg = jnp.maximum(deg, 1.0).reshape(n_pad, 1)

    x1 = _sage_linear(p1, deg, x_p, W_l1, W_r1, b_l1.reshape(1, h), n_pad, br)

    (p2,) = _make_sc_scatter(n_pad, e_pad, h, False)(x1, src, dst, z128, rix)

    ps, cnt = _pool_stage(p2, deg, x1, batch3d, W_l2, W_r2,
                          b_l2.reshape(1, h), W_blk[:h], W_blk[h:],
                          b_blk.reshape(1, h), n_pad, br)

    return _head(ps, cnt, W1, b1.reshape(1, h), W2, b2.reshape(1, W2.shape[1]))
